# baseline (speedup 1.0000x reference)
import functools

import jax
import jax.numpy as jnp
from jax.experimental import pallas as pl
from jax.experimental.pallas import tpu as pltpu

_N = 500000
_C = 100
_NBINS = 10
_BLK = 4000
_BN_ROWS = 5000
_BN_COLS = 100
_BN_BLK = 1000


def _rows_kernel(x_ref, tgt_ref, conf_ref, acc_ref):
    x = x_ref[...]
    m = jnp.max(x, axis=1, keepdims=True)
    e = jnp.exp(x - m)
    s = jnp.sum(e, axis=1, keepdims=True)
    conf_ref[...] = 1.0 / s
    iota = jax.lax.broadcasted_iota(jnp.int32, x.shape, 1)
    match = jnp.where((x == m) & (iota == tgt_ref[...]), 1.0, 0.0)
    acc_ref[...] = jnp.max(match, axis=1, keepdims=True)


def _hist_kernel(bounds_ref, conf_ref, accv_ref, out_ref, cnt_ref, sum_ref, *, nsteps):
    step = pl.program_id(0)

    @pl.when(step == 0)
    def _init():
        cnt_ref[...] = jnp.zeros_like(cnt_ref)
        sum_ref[...] = jnp.zeros_like(sum_ref)

    c = conf_ref[...]
    a = accv_ref[...]
    for i in range(_NBINS):
        lo = bounds_ref[0, i]
        hi = bounds_ref[1, i]
        inside = (c > lo) & (c <= hi)
        insf = inside.astype(jnp.float32)
        cnt_ref[i : i + 1, :] += jnp.sum(insf, axis=0, keepdims=True)
        sum_ref[i : i + 1, :] += jnp.sum(jnp.where(inside, a, 0.0), axis=0, keepdims=True)

    @pl.when(step == nsteps - 1)
    def _fin():
        cr = jnp.sum(cnt_ref[...], axis=1, keepdims=True)
        ar = jnp.sum(sum_ref[...], axis=1, keepdims=True)
        out_ref[...] = jnp.where(cr > 0, ar / jnp.maximum(cr, 1.0), 0.0)


@jax.jit
def kernel(logits, targets):
    tgt = targets.astype(jnp.int32).reshape(_N, 1)
    nsteps_a = _N // _BLK

    conf, accv = pl.pallas_call(
        _rows_kernel,
        grid=(nsteps_a,),
        in_specs=[
            pl.BlockSpec((_BLK, _C), lambda i: (i, 0)),
            pl.BlockSpec((_BLK, 1), lambda i: (i, 0)),
        ],
        out_specs=[
            pl.BlockSpec((_BLK, 1), lambda i: (i, 0)),
            pl.BlockSpec((_BLK, 1), lambda i: (i, 0)),
        ],
        out_shape=[
            jax.ShapeDtypeStruct((_N, 1), jnp.float32),
            jax.ShapeDtypeStruct((_N, 1), jnp.float32),
        ],
    )(logits, tgt)

    interval = jnp.linspace(0.0, 1.0, _NBINS + 1)
    bounds = jnp.zeros((2, _NBINS), jnp.float32)
    bounds = bounds.at[0, :].set(interval[:-1]).at[1, :].set(interval[1:])

    nsteps_b = _BN_ROWS // _BN_BLK
    out = pl.pallas_call(
        functools.partial(_hist_kernel, nsteps=nsteps_b),
        grid=(nsteps_b,),
        in_specs=[
            pl.BlockSpec(memory_space=pltpu.SMEM),
            pl.BlockSpec((_BN_BLK, _BN_COLS), lambda i: (i, 0)),
            pl.BlockSpec((_BN_BLK, _BN_COLS), lambda i: (i, 0)),
        ],
        out_specs=pl.BlockSpec((16, 1), lambda i: (0, 0)),
        out_shape=jax.ShapeDtypeStruct((16, 1), jnp.float32),
        scratch_shapes=[
            pltpu.VMEM((16, _BN_COLS), jnp.float32),
            pltpu.VMEM((16, _BN_COLS), jnp.float32),
        ],
    )(bounds, conf.reshape(_BN_ROWS, _BN_COLS), accv.reshape(_BN_ROWS, _BN_COLS))

    return out[:_NBINS, :]


# speedup vs baseline: 2.3869x; 2.3869x over previous
import functools

import jax
import jax.numpy as jnp
from jax.experimental import pallas as pl
from jax.experimental.pallas import tpu as pltpu

_N = 500000
_C = 100
_NBINS = 10
_BLK = 4000
_NBLKA = _N // _BLK
_BN_ROWS = 5000
_BN_COLS = 100
_BN_BLK = 1000


def _rows_kernel(x_ref, tgt_ref, conf_ref, acc_ref):
    xt = jnp.swapaxes(x_ref[...], 0, 1)
    m = jnp.max(xt, axis=0, keepdims=True)
    e = jnp.exp(xt - m)
    s = jnp.sum(e, axis=0, keepdims=True)
    conf_ref[0] = 1.0 / s
    iota = jax.lax.broadcasted_iota(jnp.int32, xt.shape, 0)
    tval = jnp.max(jnp.where(iota == tgt_ref[0], xt, -jnp.inf), axis=0, keepdims=True)
    acc_ref[0] = (tval == m).astype(jnp.float32)


def _hist_kernel(bounds_ref, conf_ref, accv_ref, out_ref, cnt_ref, sum_ref, *, nsteps):
    step = pl.program_id(0)

    @pl.when(step == 0)
    def _init():
        cnt_ref[...] = jnp.zeros_like(cnt_ref)
        sum_ref[...] = jnp.zeros_like(sum_ref)

    c = conf_ref[...]
    a = accv_ref[...]
    for i in range(_NBINS):
        lo = bounds_ref[0, i]
        hi = bounds_ref[1, i]
        inside = (c > lo) & (c <= hi)
        insf = inside.astype(jnp.float32)
        cnt_ref[i : i + 1, :] += jnp.sum(insf, axis=0, keepdims=True)
        sum_ref[i : i + 1, :] += jnp.sum(jnp.where(inside, a, 0.0), axis=0, keepdims=True)

    @pl.when(step == nsteps - 1)
    def _fin():
        cr = jnp.sum(cnt_ref[...], axis=1, keepdims=True)
        ar = jnp.sum(sum_ref[...], axis=1, keepdims=True)
        out_ref[...] = jnp.where(cr > 0, ar / jnp.maximum(cr, 1.0), 0.0)


@jax.jit
def kernel(logits, targets):
    tgt = targets.astype(jnp.int32).reshape(_NBLKA, 1, _BLK)

    conf, accv = pl.pallas_call(
        _rows_kernel,
        grid=(_NBLKA,),
        in_specs=[
            pl.BlockSpec((_BLK, _C), lambda i: (i, 0)),
            pl.BlockSpec((1, 1, _BLK), lambda i: (i, 0, 0)),
        ],
        out_specs=[
            pl.BlockSpec((1, 1, _BLK), lambda i: (i, 0, 0)),
            pl.BlockSpec((1, 1, _BLK), lambda i: (i, 0, 0)),
        ],
        out_shape=[
            jax.ShapeDtypeStruct((_NBLKA, 1, _BLK), jnp.float32),
            jax.ShapeDtypeStruct((_NBLKA, 1, _BLK), jnp.float32),
        ],
    )(logits, tgt)

    interval = jnp.linspace(0.0, 1.0, _NBINS + 1)
    bounds = jnp.zeros((2, _NBINS), jnp.float32)
    bounds = bounds.at[0, :].set(interval[:-1]).at[1, :].set(interval[1:])

    nsteps_b = _BN_ROWS // _BN_BLK
    out = pl.pallas_call(
        functools.partial(_hist_kernel, nsteps=nsteps_b),
        grid=(nsteps_b,),
        in_specs=[
            pl.BlockSpec(memory_space=pltpu.SMEM),
            pl.BlockSpec((_BN_BLK, _BN_COLS), lambda i: (i, 0)),
            pl.BlockSpec((_BN_BLK, _BN_COLS), lambda i: (i, 0)),
        ],
        out_specs=pl.BlockSpec((16, 1), lambda i: (0, 0)),
        out_shape=jax.ShapeDtypeStruct((16, 1), jnp.float32),
        scratch_shapes=[
            pltpu.VMEM((16, _BN_COLS), jnp.float32),
            pltpu.VMEM((16, _BN_COLS), jnp.float32),
        ],
    )(bounds, conf.reshape(_BN_ROWS, _BN_COLS), accv.reshape(_BN_ROWS, _BN_COLS))

    return out[:_NBINS, :]


# speedup vs baseline: 2.6995x; 1.1310x over previous
import functools

import jax
import jax.numpy as jnp
from jax.experimental import pallas as pl
from jax.experimental.pallas import tpu as pltpu

_N = 500000
_C = 100
_NBINS = 10
_BLK = 10000
_NBLKA = _N // _BLK
_BN_ROWS = 5000
_BN_COLS = 100
_BN_BLK = 1000


def _rows_kernel(x_ref, tgt_ref, conf_ref, acc_ref):
    xt = jnp.swapaxes(x_ref[...], 0, 1)
    m = jnp.max(xt, axis=0, keepdims=True)
    e = jnp.exp(xt - m)
    s = jnp.sum(e, axis=0, keepdims=True)
    conf_ref[0] = 1.0 / s
    iota = jax.lax.broadcasted_iota(jnp.int32, xt.shape, 0)
    tval = jnp.max(jnp.where(iota == tgt_ref[0], xt, -jnp.inf), axis=0, keepdims=True)
    acc_ref[0] = (tval == m).astype(jnp.float32)


def _hist_kernel(bounds_ref, conf_ref, accv_ref, out_ref, cnt_ref, sum_ref, *, nsteps):
    step = pl.program_id(0)

    @pl.when(step == 0)
    def _init():
        cnt_ref[...] = jnp.zeros_like(cnt_ref)
        sum_ref[...] = jnp.zeros_like(sum_ref)

    c = conf_ref[...]
    a = accv_ref[...]
    for i in range(_NBINS):
        lo = bounds_ref[0, i]
        hi = bounds_ref[1, i]
        inside = (c > lo) & (c <= hi)
        insf = inside.astype(jnp.float32)
        cnt_ref[i : i + 1, :] += jnp.sum(insf, axis=0, keepdims=True)
        sum_ref[i : i + 1, :] += jnp.sum(jnp.where(inside, a, 0.0), axis=0, keepdims=True)

    @pl.when(step == nsteps - 1)
    def _fin():
        cr = jnp.sum(cnt_ref[...], axis=1, keepdims=True)
        ar = jnp.sum(sum_ref[...], axis=1, keepdims=True)
        out_ref[...] = jnp.where(cr > 0, ar / jnp.maximum(cr, 1.0), 0.0)


@jax.jit
def kernel(logits, targets):
    tgt = targets.astype(jnp.int32).reshape(_NBLKA, 1, _BLK)

    conf, accv = pl.pallas_call(
        _rows_kernel,
        grid=(_NBLKA,),
        in_specs=[
            pl.BlockSpec((_BLK, _C), lambda i: (i, 0)),
            pl.BlockSpec((1, 1, _BLK), lambda i: (i, 0, 0)),
        ],
        out_specs=[
            pl.BlockSpec((1, 1, _BLK), lambda i: (i, 0, 0)),
            pl.BlockSpec((1, 1, _BLK), lambda i: (i, 0, 0)),
        ],
        out_shape=[
            jax.ShapeDtypeStruct((_NBLKA, 1, _BLK), jnp.float32),
            jax.ShapeDtypeStruct((_NBLKA, 1, _BLK), jnp.float32),
        ],
    )(logits, tgt)

    interval = jnp.linspace(0.0, 1.0, _NBINS + 1)
    bounds = jnp.zeros((2, _NBINS), jnp.float32)
    bounds = bounds.at[0, :].set(interval[:-1]).at[1, :].set(interval[1:])

    nsteps_b = _BN_ROWS // _BN_BLK
    out = pl.pallas_call(
        functools.partial(_hist_kernel, nsteps=nsteps_b),
        grid=(nsteps_b,),
        in_specs=[
            pl.BlockSpec(memory_space=pltpu.SMEM),
            pl.BlockSpec((_BN_BLK, _BN_COLS), lambda i: (i, 0)),
            pl.BlockSpec((_BN_BLK, _BN_COLS), lambda i: (i, 0)),
        ],
        out_specs=pl.BlockSpec((16, 1), lambda i: (0, 0)),
        out_shape=jax.ShapeDtypeStruct((16, 1), jnp.float32),
        scratch_shapes=[
            pltpu.VMEM((16, _BN_COLS), jnp.float32),
            pltpu.VMEM((16, _BN_COLS), jnp.float32),
        ],
    )(bounds, conf.reshape(_BN_ROWS, _BN_COLS), accv.reshape(_BN_ROWS, _BN_COLS))

    return out[:_NBINS, :]


# speedup vs baseline: 2.8022x; 1.0380x over previous
import functools

import jax
import jax.numpy as jnp
from jax.experimental import pallas as pl
from jax.experimental.pallas import tpu as pltpu

_N = 500000
_C = 100
_NBINS = 10
_BLK = 20000
_NBLKA = _N // _BLK
_BN_ROWS = 5000
_BN_COLS = 100
_BN_BLK = 1000


def _rows_kernel(x_ref, tgt_ref, conf_ref, acc_ref):
    xt = jnp.swapaxes(x_ref[...], 0, 1)
    m = jnp.max(xt, axis=0, keepdims=True)
    e = jnp.exp(xt - m)
    s = jnp.sum(e, axis=0, keepdims=True)
    conf_ref[0] = 1.0 / s
    iota = jax.lax.broadcasted_iota(jnp.int32, xt.shape, 0)
    tval = jnp.max(jnp.where(iota == tgt_ref[0], xt, -jnp.inf), axis=0, keepdims=True)
    acc_ref[0] = (tval == m).astype(jnp.float32)


def _hist_kernel(bounds_ref, conf_ref, accv_ref, out_ref, cnt_ref, sum_ref, *, nsteps):
    step = pl.program_id(0)

    @pl.when(step == 0)
    def _init():
        cnt_ref[...] = jnp.zeros_like(cnt_ref)
        sum_ref[...] = jnp.zeros_like(sum_ref)

    c = conf_ref[...]
    a = accv_ref[...]
    for i in range(_NBINS):
        lo = bounds_ref[0, i]
        hi = bounds_ref[1, i]
        inside = (c > lo) & (c <= hi)
        insf = inside.astype(jnp.float32)
        cnt_ref[i : i + 1, :] += jnp.sum(insf, axis=0, keepdims=True)
        sum_ref[i : i + 1, :] += jnp.sum(jnp.where(inside, a, 0.0), axis=0, keepdims=True)

    @pl.when(step == nsteps - 1)
    def _fin():
        cr = jnp.sum(cnt_ref[...], axis=1, keepdims=True)
        ar = jnp.sum(sum_ref[...], axis=1, keepdims=True)
        out_ref[...] = jnp.where(cr > 0, ar / jnp.maximum(cr, 1.0), 0.0)


@jax.jit
def kernel(logits, targets):
    tgt = targets.astype(jnp.int32).reshape(_NBLKA, 1, _BLK)

    conf, accv = pl.pallas_call(
        _rows_kernel,
        grid=(_NBLKA,),
        in_specs=[
            pl.BlockSpec((_BLK, _C), lambda i: (i, 0)),
            pl.BlockSpec((1, 1, _BLK), lambda i: (i, 0, 0)),
        ],
        out_specs=[
            pl.BlockSpec((1, 1, _BLK), lambda i: (i, 0, 0)),
            pl.BlockSpec((1, 1, _BLK), lambda i: (i, 0, 0)),
        ],
        out_shape=[
            jax.ShapeDtypeStruct((_NBLKA, 1, _BLK), jnp.float32),
            jax.ShapeDtypeStruct((_NBLKA, 1, _BLK), jnp.float32),
        ],
    )(logits, tgt)

    interval = jnp.linspace(0.0, 1.0, _NBINS + 1)
    bounds = jnp.zeros((2, _NBINS), jnp.float32)
    bounds = bounds.at[0, :].set(interval[:-1]).at[1, :].set(interval[1:])

    nsteps_b = _BN_ROWS // _BN_BLK
    out = pl.pallas_call(
        functools.partial(_hist_kernel, nsteps=nsteps_b),
        grid=(nsteps_b,),
        in_specs=[
            pl.BlockSpec(memory_space=pltpu.SMEM),
            pl.BlockSpec((_BN_BLK, _BN_COLS), lambda i: (i, 0)),
            pl.BlockSpec((_BN_BLK, _BN_COLS), lambda i: (i, 0)),
        ],
        out_specs=pl.BlockSpec((16, 1), lambda i: (0, 0)),
        out_shape=jax.ShapeDtypeStruct((16, 1), jnp.float32),
        scratch_shapes=[
            pltpu.VMEM((16, _BN_COLS), jnp.float32),
            pltpu.VMEM((16, _BN_COLS), jnp.float32),
        ],
    )(bounds, conf.reshape(_BN_ROWS, _BN_COLS), accv.reshape(_BN_ROWS, _BN_COLS))

    return out[:_NBINS, :]
